# Initial kernel scaffold; baseline (speedup 1.0000x reference)
import jax
import jax.numpy as jnp
from jax.experimental import pallas as pl


def kernel(team_offense, team_defense, conf_offense, conf_defense, winner_team_id, loser_team_id, winner_conf_id, loser_conf_id, winner_location, loser_location, W1, b1, W2, b2, affine_w, affine_b, home_w):
    raise NotImplementedError("write your pallas kernel here")


# speedup vs baseline: 1.8218x; 1.8218x over previous
import functools

import jax
import jax.numpy as jnp
from jax import lax
from jax.experimental import pallas as pl
from jax.experimental.pallas import tpu as pltpu
from jax.experimental.pallas import tpu_sc as plsc

N_TEAMS = 100000
N_CONFS = 1000
D = 128
B = 16384

NC = 2
NS = 16
NW = NC * NS
B_PER_W = B // NW
CHUNK = 128
N_CHUNKS = B_PER_W // CHUNK


def _sc_gather_body(team_off, team_def, conf_off, conf_def,
                    wt_ids, lt_ids, wc_ids, lc_ids,
                    t_wo, t_wd, t_lo, t_ld, c_wo, c_wd, c_lo, c_ld,
                    idx_v, rows_v, sem):
    wid = lax.axis_index("s") * NC + lax.axis_index("c")
    base = wid * B_PER_W

    def gathers(ids_hbm, tbl_a, out_a, tbl_b, out_b, off):
        pltpu.sync_copy(ids_hbm.at[pl.ds(off, CHUNK)], idx_v)
        pltpu.async_copy(tbl_a.at[idx_v], rows_v, sem).wait()
        pltpu.sync_copy(rows_v, out_a.at[pl.ds(off, CHUNK)])
        pltpu.async_copy(tbl_b.at[idx_v], rows_v, sem).wait()
        pltpu.sync_copy(rows_v, out_b.at[pl.ds(off, CHUNK)])

    for chunk in range(N_CHUNKS):
        off = base + chunk * CHUNK
        gathers(wt_ids, team_off, t_wo, team_def, t_wd, off)
        gathers(lt_ids, team_off, t_lo, team_def, t_ld, off)
        gathers(wc_ids, conf_off, c_wo, conf_def, c_wd, off)
        gathers(lc_ids, conf_off, c_lo, conf_def, c_ld, off)


def _sc_gather(team_off, team_def, conf_off, conf_def,
               wt_ids, lt_ids, wc_ids, lc_ids):
    out = jax.ShapeDtypeStruct((B, D), jnp.float32)
    mesh = plsc.VectorSubcoreMesh(core_axis_name="c", subcore_axis_name="s")
    return pl.kernel(
        _sc_gather_body,
        out_type=[out] * 8,
        mesh=mesh,
        scratch_types=[
            pltpu.VMEM((CHUNK,), jnp.int32),
            pltpu.VMEM((CHUNK, D), jnp.float32),
            pltpu.SemaphoreType.DMA,
        ],
    )(team_off, team_def, conf_off, conf_def, wt_ids, lt_ids, wc_ids, lc_ids)


BM = 512


def _tc_mlp_body(t_wo, c_wo, t_wd, c_wd, t_lo, c_lo, t_ld, c_ld,
                 wloc, lloc, W1, b1, W2, b2, aw, ab, hw,
                 wscore, lscore):
    wo = t_wo[...] + c_wo[...]
    wd = t_wd[...] + c_wd[...]
    lo = t_lo[...] + c_lo[...]
    ld = t_ld[...] + c_ld[...]
    W1a = W1[:D, :]
    W1b = W1[D:, :]
    bias = b1[...]
    h_w = jnp.maximum(
        jnp.dot(wo, W1a, preferred_element_type=jnp.float32)
        + jnp.dot(ld, W1b, preferred_element_type=jnp.float32) + bias, 0.0)
    h_l = jnp.maximum(
        jnp.dot(lo, W1a, preferred_element_type=jnp.float32)
        + jnp.dot(wd, W1b, preferred_element_type=jnp.float32) + bias, 0.0)
    ws = jnp.dot(h_w, W2[...], preferred_element_type=jnp.float32) + b2[0, 0]
    ls = jnp.dot(h_l, W2[...], preferred_element_type=jnp.float32) + b2[0, 0]
    a_w = aw[0, 0]
    a_b = ab[0, 0]
    h_f = hw[0, 0]
    wscore[...] = ws * a_w + a_b + wloc[...] * h_f
    lscore[...] = ls * a_w + a_b + lloc[...] * h_f


def _tc_mlp(t_wo, c_wo, t_wd, c_wd, t_lo, c_lo, t_ld, c_ld,
            wloc, lloc, W1, b1, W2, b2, aw, ab, hw):
    grid = (B // BM,)
    row_spec = pl.BlockSpec((BM, D), lambda i: (i, 0))
    col_spec = pl.BlockSpec((BM, 1), lambda i: (i, 0))
    full = lambda shape: pl.BlockSpec(shape, lambda i: (0,) * len(shape))
    return pl.pallas_call(
        _tc_mlp_body,
        grid=grid,
        in_specs=[row_spec] * 8 + [col_spec] * 2 + [
            full((2 * D, D)), full((1, D)), full((D, 1)),
            full((1, 1)), full((1, 1)), full((1, 1)), full((1, 1)),
        ],
        out_specs=[col_spec, col_spec],
        out_shape=[jax.ShapeDtypeStruct((B, 1), jnp.float32)] * 2,
    )(t_wo, c_wo, t_wd, c_wd, t_lo, c_lo, t_ld, c_ld,
      wloc, lloc, W1, b1, W2, b2, aw, ab, hw)


def kernel(team_offense, team_defense, conf_offense, conf_defense,
           winner_team_id, loser_team_id, winner_conf_id, loser_conf_id,
           winner_location, loser_location,
           W1, b1, W2, b2, affine_w, affine_b, home_w):
    t_wo, t_wd, t_lo, t_ld, c_wo, c_wd, c_lo, c_ld = _sc_gather(
        team_offense, team_defense, conf_offense, conf_defense,
        winner_team_id.astype(jnp.int32), loser_team_id.astype(jnp.int32),
        winner_conf_id.astype(jnp.int32), loser_conf_id.astype(jnp.int32))
    wscore, lscore = _tc_mlp(
        t_wo, c_wo, t_wd, c_wd, t_lo, c_lo, t_ld, c_ld,
        winner_location, loser_location,
        W1, b1.reshape(1, D), W2, b2.reshape(1, 1),
        affine_w, affine_b.reshape(1, 1), home_w)
    return (wscore, lscore)


# speedup vs baseline: 2.2056x; 1.2106x over previous
import functools

import jax
import jax.numpy as jnp
from jax import lax
from jax.experimental import pallas as pl
from jax.experimental.pallas import tpu as pltpu
from jax.experimental.pallas import tpu_sc as plsc

N_TEAMS = 100000
N_CONFS = 1000
D = 128
B = 16384

NC = 2
NS = 16
NW = NC * NS
B_PER_W = B // NW
CHUNK = 128
N_CHUNKS = B_PER_W // CHUNK


N_BUF = 6


def _sc_gather_body(team_off, team_def, conf_off, conf_def,
                    wt_ids, lt_ids, wc_ids, lc_ids,
                    t_wo, t_wd, t_lo, t_ld, c_wo, c_wd, c_lo, c_ld,
                    *scratch):
    rows = scratch[:N_BUF]
    idxs = scratch[N_BUF:N_BUF + 4]
    g_sems = scratch[N_BUF + 4:2 * N_BUF + 4]
    wb_sems = scratch[2 * N_BUF + 4:3 * N_BUF + 4]
    idx_sem = scratch[3 * N_BUF + 4]

    wid = lax.axis_index("s") * NC + lax.axis_index("c")
    base = wid * B_PER_W

    idescs = [
        pltpu.async_copy(ids.at[pl.ds(wid * N_CHUNKS, N_CHUNKS)], idx, idx_sem)
        for ids, idx in zip((wt_ids, lt_ids, wc_ids, lc_ids), idxs)
    ]
    for d in idescs:
        d.wait()

    ops = []
    for c in range(N_CHUNKS):
        off = base + c * CHUNK
        for idx, t_a, o_a, t_b, o_b in (
                (idxs[0], team_off, t_wo, team_def, t_wd),
                (idxs[1], team_off, t_lo, team_def, t_ld),
                (idxs[2], conf_off, c_wo, conf_def, c_wd),
                (idxs[3], conf_off, c_lo, conf_def, c_ld)):
            ops.append((idx.at[c], t_a, o_a, off))
            ops.append((idx.at[c], t_b, o_b, off))

    n_ops = len(ops)
    g_descs = [None] * n_ops
    wb_descs = [None] * n_ops
    for i, (idx, tbl, out, off) in enumerate(ops):
        b = i % N_BUF
        if i >= N_BUF:
            wb_descs[i - N_BUF].wait()
        g_descs[i] = pltpu.async_copy(tbl.at[idx], rows[b], g_sems[b])
        if i >= 1:
            pidx, ptbl, pout, poff = ops[i - 1]
            pb = (i - 1) % N_BUF
            g_descs[i - 1].wait()
            wb_descs[i - 1] = pltpu.async_copy(
                rows[pb], pout.at[pl.ds(poff, CHUNK)], wb_sems[pb])
    g_descs[n_ops - 1].wait()
    lidx, ltbl, lout, loff = ops[n_ops - 1]
    wb_descs[n_ops - 1] = pltpu.async_copy(
        rows[(n_ops - 1) % N_BUF], lout.at[pl.ds(loff, CHUNK)],
        wb_sems[(n_ops - 1) % N_BUF])
    for j in range(n_ops - N_BUF, n_ops):
        wb_descs[j].wait()


def _sc_gather(team_off, team_def, conf_off, conf_def,
               wt_ids, lt_ids, wc_ids, lc_ids):
    out = jax.ShapeDtypeStruct((B, D), jnp.float32)
    mesh = plsc.VectorSubcoreMesh(core_axis_name="c", subcore_axis_name="s")
    return pl.kernel(
        _sc_gather_body,
        out_type=[out] * 8,
        mesh=mesh,
        scratch_types=(
            [pltpu.VMEM((CHUNK, D), jnp.float32)] * N_BUF
            + [pltpu.VMEM((N_CHUNKS, CHUNK), jnp.int32)] * 4
            + [pltpu.SemaphoreType.DMA] * (2 * N_BUF + 1)
        ),
    )(team_off, team_def, conf_off, conf_def, wt_ids, lt_ids, wc_ids, lc_ids)


BM = 512


def _tc_mlp_body(t_wo, c_wo, t_wd, c_wd, t_lo, c_lo, t_ld, c_ld,
                 wloc, lloc, W1, b1, W2, b2, aw, ab, hw,
                 wscore, lscore):
    wo = t_wo[...] + c_wo[...]
    wd = t_wd[...] + c_wd[...]
    lo = t_lo[...] + c_lo[...]
    ld = t_ld[...] + c_ld[...]
    W1a = W1[:D, :]
    W1b = W1[D:, :]
    bias = b1[...]
    h_w = jnp.maximum(
        jnp.dot(wo, W1a, preferred_element_type=jnp.float32)
        + jnp.dot(ld, W1b, preferred_element_type=jnp.float32) + bias, 0.0)
    h_l = jnp.maximum(
        jnp.dot(lo, W1a, preferred_element_type=jnp.float32)
        + jnp.dot(wd, W1b, preferred_element_type=jnp.float32) + bias, 0.0)
    ws = jnp.dot(h_w, W2[...], preferred_element_type=jnp.float32) + b2[0, 0]
    ls = jnp.dot(h_l, W2[...], preferred_element_type=jnp.float32) + b2[0, 0]
    a_w = aw[0, 0]
    a_b = ab[0, 0]
    h_f = hw[0, 0]
    wscore[...] = ws * a_w + a_b + wloc[...] * h_f
    lscore[...] = ls * a_w + a_b + lloc[...] * h_f


def _tc_mlp(t_wo, c_wo, t_wd, c_wd, t_lo, c_lo, t_ld, c_ld,
            wloc, lloc, W1, b1, W2, b2, aw, ab, hw):
    grid = (B // BM,)
    row_spec = pl.BlockSpec((BM, D), lambda i: (i, 0))
    col_spec = pl.BlockSpec((BM, 1), lambda i: (i, 0))
    full = lambda shape: pl.BlockSpec(shape, lambda i: (0,) * len(shape))
    return pl.pallas_call(
        _tc_mlp_body,
        grid=grid,
        in_specs=[row_spec] * 8 + [col_spec] * 2 + [
            full((2 * D, D)), full((1, D)), full((D, 1)),
            full((1, 1)), full((1, 1)), full((1, 1)), full((1, 1)),
        ],
        out_specs=[col_spec, col_spec],
        out_shape=[jax.ShapeDtypeStruct((B, 1), jnp.float32)] * 2,
    )(t_wo, c_wo, t_wd, c_wd, t_lo, c_lo, t_ld, c_ld,
      wloc, lloc, W1, b1, W2, b2, aw, ab, hw)


def kernel(team_offense, team_defense, conf_offense, conf_defense,
           winner_team_id, loser_team_id, winner_conf_id, loser_conf_id,
           winner_location, loser_location,
           W1, b1, W2, b2, affine_w, affine_b, home_w):
    rs = lambda x: x.astype(jnp.int32).reshape(B // CHUNK, CHUNK)
    t_wo, t_wd, t_lo, t_ld, c_wo, c_wd, c_lo, c_ld = _sc_gather(
        team_offense, team_defense, conf_offense, conf_defense,
        rs(winner_team_id), rs(loser_team_id),
        rs(winner_conf_id), rs(loser_conf_id))
    wscore, lscore = _tc_mlp(
        t_wo, c_wo, t_wd, c_wd, t_lo, c_lo, t_ld, c_ld,
        winner_location, loser_location,
        W1, b1.reshape(1, D), W2, b2.reshape(1, 1),
        affine_w, affine_b.reshape(1, 1), home_w)
    return (wscore, lscore)
